# Initial kernel scaffold; baseline (speedup 1.0000x reference)
import jax
import jax.numpy as jnp
from jax.experimental import pallas as pl


def kernel(x, edge_index, W0, b0, Wres0, bres0, W1, b1, Wres1, bres1, Wg, bg, gamma, beta, Wc1, bc1, Wc2, bc2):
    raise NotImplementedError("write your pallas kernel here")


# speedup vs baseline: 15.8520x; 15.8520x over previous
import functools

import jax
import jax.numpy as jnp
from jax import lax
from jax.experimental import pallas as pl
from jax.experimental.pallas import tpu as pltpu
from jax.experimental.pallas import tpu_sc as plsc

N = 10000
D = 128
H = 128
HD = D // 2
NC = 2
NS = 16
NW = NC * NS
CH = 128
NPAD = 10240
PADROWS = NPAD - N
ROWS_PER_SUB = NPAD // NS


def _mesh():
    return plsc.VectorSubcoreMesh(core_axis_name="c", subcore_axis_name="s",
                                  num_cores=NC, num_subcores=NS)


def _sc_degrees(nch):
    @functools.partial(
        pl.kernel,
        out_type=jax.ShapeDtypeStruct((NC, 2, NPAD, 16), jnp.float32),
        mesh=_mesh(),
        scratch_types=[
            pltpu.VMEM((nch, CH), jnp.int32),
            pltpu.VMEM((nch, CH), jnp.int32),
            pltpu.VMEM((CH, 16), jnp.float32),
            pltpu.VMEM_SHARED((NPAD, 16), jnp.float32),
            pltpu.VMEM_SHARED((NPAD, 16), jnp.float32),
        ],
    )
    def deg_kernel(src_hbm, dst_hbm, ones_hbm, zeros_hbm, out_hbm,
                   sidx, didx, ones_v, dsrc_sh, ddst_sh):
        cid = lax.axis_index("c")
        sid = lax.axis_index("s")
        wid = sid * NC + cid
        rbase = sid * ROWS_PER_SUB
        pltpu.sync_copy(zeros_hbm.at[pl.ds(rbase, ROWS_PER_SUB)],
                        dsrc_sh.at[pl.ds(rbase, ROWS_PER_SUB)])
        pltpu.sync_copy(zeros_hbm.at[pl.ds(rbase, ROWS_PER_SUB)],
                        ddst_sh.at[pl.ds(rbase, ROWS_PER_SUB)])
        pltpu.sync_copy(ones_hbm, ones_v)
        pltpu.sync_copy(src_hbm.at[wid], sidx)
        pltpu.sync_copy(dst_hbm.at[wid], didx)
        plsc.subcore_barrier()

        @pl.loop(0, nch)
        def _(j):
            pltpu.sync_copy(ones_v, dsrc_sh.at[sidx.at[j]], add=True)
            pltpu.sync_copy(ones_v, ddst_sh.at[didx.at[j]], add=True)

        plsc.subcore_barrier()
        pltpu.sync_copy(dsrc_sh.at[pl.ds(rbase, ROWS_PER_SUB)],
                        out_hbm.at[cid, 0, pl.ds(rbase, ROWS_PER_SUB)])
        pltpu.sync_copy(ddst_sh.at[pl.ds(rbase, ROWS_PER_SUB)],
                        out_hbm.at[cid, 1, pl.ds(rbase, ROWS_PER_SUB)])

    return deg_kernel


def _sc_aggregate(nch, nblk, blk):
    @functools.partial(
        pl.kernel,
        out_type=jax.ShapeDtypeStruct((NC, NPAD, D), jnp.float32),
        mesh=_mesh(),
        scratch_types=[
            pltpu.VMEM((blk, CH), jnp.int32),
            pltpu.VMEM((blk, CH), jnp.int32),
            pltpu.VMEM((CH, D), jnp.float32),
            pltpu.VMEM((CH, D), jnp.float32),
            pltpu.VMEM_SHARED((NPAD, D), jnp.float32),
            pltpu.SemaphoreType.DMA,
            pltpu.SemaphoreType.DMA,
        ],
    )
    def agg_kernel(t_hbm, src_hbm, dst_hbm, zeros_hbm, out_hbm,
                   sidx, didx, rows0, rows1, agg_sh, sem0, sem1):
        cid = lax.axis_index("c")
        sid = lax.axis_index("s")
        wid = sid * NC + cid
        rbase = sid * ROWS_PER_SUB
        pltpu.sync_copy(zeros_hbm.at[pl.ds(rbase, ROWS_PER_SUB)],
                        agg_sh.at[pl.ds(rbase, ROWS_PER_SUB)])
        plsc.subcore_barrier()

        rows = (rows0, rows1)
        sems = (sem0, sem1)

        @pl.loop(0, nblk)
        def _(b):
            off = pl.multiple_of(b * blk, 8)
            pltpu.sync_copy(src_hbm.at[wid, pl.ds(off, blk)], sidx)
            pltpu.sync_copy(dst_hbm.at[wid, pl.ds(off, blk)], didx)
            pltpu.async_copy(t_hbm.at[sidx.at[0]], rows0, sem0)
            for k in range(blk):
                if k + 1 < blk:
                    pltpu.async_copy(t_hbm.at[sidx.at[k + 1]],
                                     rows[(k + 1) % 2], sems[(k + 1) % 2])
                pltpu.make_async_copy(t_hbm.at[sidx.at[k]], rows[k % 2],
                                      sems[k % 2]).wait()
                pltpu.sync_copy(rows[k % 2], agg_sh.at[didx.at[k]], add=True)

        plsc.subcore_barrier()
        pltpu.sync_copy(agg_sh.at[pl.ds(rbase, ROWS_PER_SUB)],
                        out_hbm.at[cid, pl.ds(rbase, ROWS_PER_SUB)])

    return agg_kernel


def _tc_prep(degp_ref, x_ref, xs_ref, ns_ref, nd_ref):
    dsrc = degp_ref[0, 0, :, 0:1] + degp_ref[1, 0, :, 0:1]
    ddst = degp_ref[0, 1, :, 0:1] + degp_ref[1, 1, :, 0:1]
    ns = lax.rsqrt(jnp.maximum(dsrc, 1.0))
    nd = lax.rsqrt(jnp.maximum(ddst, 1.0))
    ns_ref[...] = ns
    nd_ref[...] = nd
    xs_ref[...] = x_ref[...] * ns


def _layer_body(p_ref, h_ref, nd_ref, w_ref, b_ref, wr_ref, br_ref):
    agg = (p_ref[0] + p_ref[1]) * nd_ref[...]
    conv = jnp.maximum(
        jnp.dot(agg, w_ref[...], preferred_element_type=jnp.float32)
        + b_ref[...], 0.0)
    res = jnp.maximum(
        jnp.dot(h_ref[...], wr_ref[...], preferred_element_type=jnp.float32)
        + br_ref[...], 0.0)
    h = conv + res
    rid = lax.broadcasted_iota(jnp.int32, (NPAD, 1), 0)
    return jnp.where(rid < N, h, 0.0)


def _tc_layer(p_ref, h_ref, ns_ref, nd_ref, w_ref, b_ref, wr_ref, br_ref,
              out_h_ref, hs_ref):
    h = _layer_body(p_ref, h_ref, nd_ref, w_ref, b_ref, wr_ref, br_ref)
    out_h_ref[...] = h
    hs_ref[...] = h * ns_ref[...]


def _tc_final(p_ref, h_ref, nd_ref, w_ref, b_ref, wr_ref, br_ref,
              wg_ref, bg_ref, gamma_ref, beta_ref, wc1_ref, bc1_ref,
              wc2_ref, bc2_ref, out_ref):
    h = _layer_body(p_ref, h_ref, nd_ref, w_ref, b_ref, wr_ref, br_ref)
    logit = jnp.dot(h, wg_ref[...], preferred_element_type=jnp.float32) \
        + bg_ref[...]
    w = 1.0 / (1.0 + jnp.exp(-logit))
    gsum = jnp.sum(h * w, axis=0)
    gmax = jnp.max(h, axis=0)
    g = jnp.concatenate([gsum, gmax])[None, :]
    z = jnp.maximum(
        jnp.dot(g, wc1_ref[...], preferred_element_type=jnp.float32)
        + bc1_ref[...], 0.0)
    z = (z * float(1.0 / (1.0 + 1e-5) ** 0.5)) * gamma_ref[...] + beta_ref[...]
    zo = jnp.dot(z, wc2_ref[...], preferred_element_type=jnp.float32) \
        + bc2_ref[...]
    out_ref[...] = 1.0 / (1.0 + jnp.exp(-zo))


def _tc_call(fn, out_shapes):
    return pl.pallas_call(fn, out_shape=out_shapes)


def kernel(x, edge_index, W0, b0, Wres0, bres0, W1, b1, Wres1, bres1,
           Wg, bg, gamma, beta, Wc1, bc1, Wc2, bc2):
    e = edge_index.shape[1]
    blk = 8
    step = NW * CH
    nch = (e + step - 1) // step
    nch = ((nch + 2 * blk - 1) // (2 * blk)) * (2 * blk)
    nblk = nch // blk
    epad = nch * step

    src = edge_index[0].astype(jnp.int32)
    dst = edge_index[1].astype(jnp.int32)
    pad_idx = (N + jnp.arange(epad - e, dtype=jnp.int32) % PADROWS)
    src3 = jnp.concatenate([src, pad_idx]).reshape(NW, nch, CH)
    dst3 = jnp.concatenate([dst, pad_idx]).reshape(NW, nch, CH)

    x_pad = jnp.zeros((NPAD, D), jnp.float32).at[:N].set(x)
    zeros_nd = jnp.zeros((NPAD, D), jnp.float32)
    zeros_16 = jnp.zeros((NPAD, 16), jnp.float32)
    ones_rows = jnp.ones((CH, 16), jnp.float32)

    degp = _sc_degrees(nch)(src3, dst3, ones_rows, zeros_16)

    f32 = jnp.float32
    xs, ns, nd = _tc_call(_tc_prep, [
        jax.ShapeDtypeStruct((NPAD, D), f32),
        jax.ShapeDtypeStruct((NPAD, 1), f32),
        jax.ShapeDtypeStruct((NPAD, 1), f32),
    ])(degp, x_pad)

    agg_fn = _sc_aggregate(nch, nblk, blk)
    p0 = agg_fn(xs, src3, dst3, zeros_nd)

    h1, h1s = _tc_call(_tc_layer, [
        jax.ShapeDtypeStruct((NPAD, H), f32),
        jax.ShapeDtypeStruct((NPAD, H), f32),
    ])(p0, x_pad, ns, nd, W0, b0[None, :], Wres0, bres0[None, :])

    p1 = agg_fn(h1s, src3, dst3, zeros_nd)

    (out,) = _tc_call(_tc_final, [jax.ShapeDtypeStruct((1, 1), f32)])(
        p1, h1, nd, W1, b1[None, :], Wres1, bres1[None, :],
        Wg, bg[None, :], gamma[None, :], beta[None, :],
        Wc1, bc1[None, :], Wc2, bc2[None, :])
    return out
